# baseline (speedup 1.0000x reference)
import functools

import numpy as np
import jax
import jax.numpy as jnp
from jax import lax
from jax.experimental import pallas as pl
from jax.experimental.pallas import tpu as pltpu
from jax.experimental.pallas import tpu_sc as plsc

B = 16384
F = 26
E = 16
FEAT_DIM = 1000000
D_IN = F * E
BF = B * F
NW = 32

CPD = 32768
NBD = -(-FEAT_DIM // CPD)
PS = NBD * CPD // 128
PSW = PS * 128


def _detile_body(xt_ref, o_ref):
    o_ref[...] = xt_ref[...].reshape(8, CPD // 128, 128)


W1C = 524288
W1_ROWS = 2 * W1C // 128
W1_LIN = W1_ROWS * 128


def _w1_body(w_ref, o_ref):
    o_ref[...] = w_ref[...].reshape(W1C // 128, 128)


CV = 1664
NCH = -(-FEAT_DIM // CV)
VT = NCH * CV
_CHM, _CHX = divmod(NCH, NW)


@functools.lru_cache(maxsize=None)
def _sc_transpose_fn():
    mesh = plsc.VectorSubcoreMesh(core_axis_name="c", subcore_axis_name="s")

    @functools.partial(
        pl.kernel,
        mesh=mesh,
        compiler_params=pltpu.CompilerParams(use_tc_tiling_on_sc=False,
                                             needs_layout_passes=False),
        out_type=jax.ShapeDtypeStruct((VT * E,), jnp.float32),
        scratch_types=[
            pltpu.VMEM((E * CV,), jnp.float32),
            pltpu.VMEM((CV * E,), jnp.float32),
            pltpu.SemaphoreType.DMA,
        ],
    )
    def _sc_transpose(planes_hbm, out_hbm, strips, o16, sem):
        wid = lax.axis_index("s") * 2 + lax.axis_index("c")
        n_my = jnp.where(wid < _CHX, _CHM + 1, _CHM)
        base16 = lax.iota(jnp.int32, 16) * CV

        def chunk_body(t, carry):
            cw = wid + NW * t
            v0 = cw * CV
            handles = [
                pltpu.async_copy(planes_hbm.at[pl.ds(e * PSW + v0, CV)],
                                 strips.at[pl.ds(e * CV, CV)], sem)
                for e in range(E)
            ]
            for h in handles:
                h.wait()

            def row_body(r, c2):
                for k in range(8):
                    i = r * 8 + k
                    col = plsc.load_gather(strips, [base16 + i])
                    o16[pl.ds(i * 16, 16)] = col
                return c2

            lax.fori_loop(0, CV // 8, row_body, 0)
            pltpu.sync_copy(o16, out_hbm.at[pl.ds(v0 * E, CV * E)])
            return carry

        lax.fori_loop(0, n_my, chunk_body, 0)

    return _sc_transpose


PER_W = BF // NW
K = 1664
NCHUNK = PER_W // K


@functools.lru_cache(maxsize=None)
def _sc_gather_fn():
    mesh = plsc.VectorSubcoreMesh(core_axis_name="c", subcore_axis_name="s")

    @functools.partial(
        pl.kernel,
        mesh=mesh,
        compiler_params=pltpu.CompilerParams(use_tc_tiling_on_sc=False),
        out_type=(
            jax.ShapeDtypeStruct((BF, E), jnp.float32),
            jax.ShapeDtypeStruct((BF,), jnp.float32),
        ),
        scratch_types=[
            pltpu.VMEM((K,), jnp.int32),
            pltpu.VMEM((K,), jnp.int32),
            pltpu.VMEM((K, E), jnp.float32),
            pltpu.VMEM((K, E), jnp.float32),
            pltpu.VMEM((K,), jnp.float32),
            pltpu.VMEM((K,), jnp.float32),
            pltpu.SemaphoreType.DMA,
            pltpu.SemaphoreType.DMA,
        ],
    )
    def _sc_gather(idx_hbm, emb_hbm, w1_hbm, emb_out, w1_out,
                   idx_a, idx_b, rows_a, rows_b, w1a, w1b, sem_r, sem_w):
        wid = lax.axis_index("s") * 2 + lax.axis_index("c")
        base = wid * PER_W
        idx_bufs = (idx_a, idx_b)
        row_bufs = (rows_a, rows_b)
        w1_bufs = (w1a, w1b)

        def load_idx(c):
            pltpu.sync_copy(idx_hbm.at[pl.ds(base + c * K, K)], idx_bufs[c % 2])

        def start_gather(c):
            h1 = pltpu.async_copy(emb_hbm.at[idx_bufs[c % 2]], row_bufs[c % 2], sem_r)
            h2 = pltpu.async_copy(w1_hbm.at[idx_bufs[c % 2]], w1_bufs[c % 2], sem_w)
            return (h1, h2)

        load_idx(0)
        pend = start_gather(0)
        for c in range(NCHUNK):
            if c + 1 < NCHUNK:
                load_idx(c + 1)
            for h in pend:
                h.wait()
            if c + 1 < NCHUNK:
                pend = start_gather(c + 1)
            pltpu.sync_copy(row_bufs[c % 2], emb_out.at[pl.ds(base + c * K, K)])
            pltpu.sync_copy(w1_bufs[c % 2], w1_out.at[pl.ds(base + c * K, K)])

    return _sc_gather


_R_np = np.kron(np.eye(F, dtype=np.float32), np.ones((1, E), dtype=np.float32))
_G_np = np.tile(np.eye(E, dtype=np.float32), (F, 1))

BM = 1024
GRID = B // BM


def _tc_body(x_ref, v_ref, w1g_ref, r_ref, g_ref, w0_ref, b0_ref, w1_ref, b1_ref,
             w2_ref, b2_ref, wo_ref, scal_ref, o_ref):
    x = x_ref[...]
    v = v_ref[...]
    w1g = w1g_ref[...]
    fm_bias = scal_ref[0, 0]
    wo0 = scal_ref[0, 1]
    bo0 = scal_ref[0, 2]
    y1 = jnp.sum(w1g * v, axis=1, keepdims=True)
    vexp = jnp.dot(v, r_ref[...], preferred_element_type=jnp.float32)
    ev = x * vexp
    s = jnp.dot(ev, g_ref[...], preferred_element_type=jnp.float32)
    sq = jnp.dot(ev * ev, g_ref[...], preferred_element_type=jnp.float32)
    y2 = 0.5 * (jnp.sum(s * s, axis=1, keepdims=True)
                - jnp.sum(sq, axis=1, keepdims=True))
    yfm = y1 + y2 + fm_bias
    h = jnp.maximum(jnp.dot(x, w0_ref[...], preferred_element_type=jnp.float32)
                    + b0_ref[...], 0.0)
    h = jnp.maximum(jnp.dot(h, w1_ref[...], preferred_element_type=jnp.float32)
                    + b1_ref[...], 0.0)
    h = jnp.maximum(jnp.dot(h, w2_ref[...], preferred_element_type=jnp.float32)
                    + b2_ref[...], 0.0)
    z = yfm * wo0 + jnp.dot(h, wo_ref[...], preferred_element_type=jnp.float32) + bo0
    o_ref[...] = jax.nn.sigmoid(z)


def kernel(feat_index, feat_value, emb_W, w1, fm_bias, W0, b0, W1, b1, W2, b2, Wo, bo):
    idx_flat = feat_index.reshape(-1).astype(jnp.int32)
    planes = pl.pallas_call(
        _detile_body,
        grid=(2, NBD),
        in_specs=[pl.BlockSpec((8, CPD), lambda g, i: (g, i))],
        out_specs=pl.BlockSpec((8, CPD // 128, 128), lambda g, i: (g, i, 0)),
        out_shape=jax.ShapeDtypeStruct((16, PS, 128), jnp.float32),
    )(emb_W.T)
    w1_128 = pl.pallas_call(
        _w1_body,
        grid=(2,),
        in_specs=[pl.BlockSpec((1, W1C), lambda i: (0, i))],
        out_specs=pl.BlockSpec((W1C // 128, 128), lambda i: (i, 0)),
        out_shape=jax.ShapeDtypeStruct((W1_ROWS, 128), jnp.float32),
    )(w1.T)
    table1d = _sc_transpose_fn()(planes.reshape(16 * PS * 128))
    emb_lin = table1d.reshape(VT, E)
    emb_rows, w1g = _sc_gather_fn()(idx_flat, emb_lin, w1_128.reshape(W1_LIN))
    x = emb_rows.reshape(B, D_IN)
    w1g2 = w1g.reshape(B, F)
    scal = jnp.stack([fm_bias.astype(jnp.float32), Wo[0, 0], bo[0]]).reshape(1, 3)
    out = pl.pallas_call(
        _tc_body,
        grid=(GRID,),
        in_specs=[
            pl.BlockSpec((BM, D_IN), lambda i: (i, 0)),
            pl.BlockSpec((BM, F), lambda i: (i, 0)),
            pl.BlockSpec((BM, F), lambda i: (i, 0)),
            pl.BlockSpec((F, D_IN), lambda i: (0, 0)),
            pl.BlockSpec((D_IN, E), lambda i: (0, 0)),
            pl.BlockSpec((D_IN, 32), lambda i: (0, 0)),
            pl.BlockSpec((1, 32), lambda i: (0, 0)),
            pl.BlockSpec((32, 32), lambda i: (0, 0)),
            pl.BlockSpec((1, 32), lambda i: (0, 0)),
            pl.BlockSpec((32, 32), lambda i: (0, 0)),
            pl.BlockSpec((1, 32), lambda i: (0, 0)),
            pl.BlockSpec((32, 1), lambda i: (0, 0)),
            pl.BlockSpec((1, 3), lambda i: (0, 0)),
        ],
        out_specs=pl.BlockSpec((BM, 1), lambda i: (i, 0)),
        out_shape=jax.ShapeDtypeStruct((B, 1), jnp.float32),
    )(x, feat_value, w1g2, jnp.asarray(_R_np), jnp.asarray(_G_np),
      W0, b0.reshape(1, 32), W1, b1.reshape(1, 32), W2, b2.reshape(1, 32),
      Wo[1:, :], scal)
    return out


# speedup vs baseline: 1.3783x; 1.3783x over previous
import functools

import numpy as np
import jax
import jax.numpy as jnp
from jax import lax
from jax.experimental import pallas as pl
from jax.experimental.pallas import tpu as pltpu
from jax.experimental.pallas import tpu_sc as plsc

B = 16384
F = 26
E = 16
FEAT_DIM = 1000000
D_IN = F * E
BF = B * F
NW = 32

CPD = 32768
NBD = -(-FEAT_DIM // CPD)
PS = NBD * CPD // 128
PSW = PS * 128


def _detile_body(xt_ref, o_ref):
    o_ref[...] = xt_ref[...].reshape(8, CPD // 128, 128)


W1C = 524288
W1_ROWS = 2 * W1C // 128
W1_LIN = W1_ROWS * 128


def _w1_body(w_ref, o_ref):
    o_ref[...] = w_ref[...].reshape(W1C // 128, 128)


CV = 1664
NCH = -(-FEAT_DIM // CV)
VT = NCH * CV
_CHM, _CHX = divmod(NCH, NW)


@functools.lru_cache(maxsize=None)
def _sc_transpose_fn():
    mesh = plsc.VectorSubcoreMesh(core_axis_name="c", subcore_axis_name="s")

    @functools.partial(
        pl.kernel,
        mesh=mesh,
        compiler_params=pltpu.CompilerParams(use_tc_tiling_on_sc=False,
                                             needs_layout_passes=False),
        out_type=jax.ShapeDtypeStruct((VT * E,), jnp.float32),
        scratch_types=[
            pltpu.VMEM((E * CV,), jnp.float32),
            pltpu.VMEM((CV * E,), jnp.float32),
            pltpu.SemaphoreType.DMA,
        ],
    )
    def _sc_transpose(planes_hbm, out_hbm, strips, o16, sem):
        wid = lax.axis_index("s") * 2 + lax.axis_index("c")
        n_my = jnp.where(wid < _CHX, _CHM + 1, _CHM)
        base16 = lax.iota(jnp.int32, 16) * CV

        def chunk_body(t, carry):
            cw = wid + NW * t
            v0 = cw * CV
            handles = [
                pltpu.async_copy(planes_hbm.at[pl.ds(e * PSW + v0, CV)],
                                 strips.at[pl.ds(e * CV, CV)], sem)
                for e in range(E)
            ]
            for h in handles:
                h.wait()

            @plsc.parallel_loop(0, CV, 1, unroll=8)
            def _row(i):
                col = plsc.load_gather(strips, [base16 + i])
                o16[pl.ds(i * 16, 16)] = col
            pltpu.sync_copy(o16, out_hbm.at[pl.ds(v0 * E, CV * E)])
            return carry

        lax.fori_loop(0, n_my, chunk_body, 0)

    return _sc_transpose


PER_W = BF // NW
K = 1664
NCHUNK = PER_W // K


@functools.lru_cache(maxsize=None)
def _sc_gather_fn():
    mesh = plsc.VectorSubcoreMesh(core_axis_name="c", subcore_axis_name="s")

    @functools.partial(
        pl.kernel,
        mesh=mesh,
        compiler_params=pltpu.CompilerParams(use_tc_tiling_on_sc=False),
        out_type=(
            jax.ShapeDtypeStruct((BF, E), jnp.float32),
            jax.ShapeDtypeStruct((BF,), jnp.float32),
        ),
        scratch_types=[
            pltpu.VMEM((K,), jnp.int32),
            pltpu.VMEM((K,), jnp.int32),
            pltpu.VMEM((K, E), jnp.float32),
            pltpu.VMEM((K, E), jnp.float32),
            pltpu.VMEM((K,), jnp.float32),
            pltpu.VMEM((K,), jnp.float32),
            pltpu.SemaphoreType.DMA,
            pltpu.SemaphoreType.DMA,
        ],
    )
    def _sc_gather(idx_hbm, emb_hbm, w1_hbm, emb_out, w1_out,
                   idx_a, idx_b, rows_a, rows_b, w1a, w1b, sem_r, sem_w):
        wid = lax.axis_index("s") * 2 + lax.axis_index("c")
        base = wid * PER_W
        idx_bufs = (idx_a, idx_b)
        row_bufs = (rows_a, rows_b)
        w1_bufs = (w1a, w1b)

        def load_idx(c):
            pltpu.sync_copy(idx_hbm.at[pl.ds(base + c * K, K)], idx_bufs[c % 2])

        def start_gather(c):
            h1 = pltpu.async_copy(emb_hbm.at[idx_bufs[c % 2]], row_bufs[c % 2], sem_r)
            h2 = pltpu.async_copy(w1_hbm.at[idx_bufs[c % 2]], w1_bufs[c % 2], sem_w)
            return (h1, h2)

        load_idx(0)
        pend = start_gather(0)
        for c in range(NCHUNK):
            if c + 1 < NCHUNK:
                load_idx(c + 1)
            for h in pend:
                h.wait()
            if c + 1 < NCHUNK:
                pend = start_gather(c + 1)
            pltpu.sync_copy(row_bufs[c % 2], emb_out.at[pl.ds(base + c * K, K)])
            pltpu.sync_copy(w1_bufs[c % 2], w1_out.at[pl.ds(base + c * K, K)])

    return _sc_gather


_R_np = np.kron(np.eye(F, dtype=np.float32), np.ones((1, E), dtype=np.float32))
_G_np = np.tile(np.eye(E, dtype=np.float32), (F, 1))

BM = 1024
GRID = B // BM


def _tc_body(x_ref, v_ref, w1g_ref, r_ref, g_ref, w0_ref, b0_ref, w1_ref, b1_ref,
             w2_ref, b2_ref, wo_ref, scal_ref, o_ref):
    x = x_ref[...]
    v = v_ref[...]
    w1g = w1g_ref[...]
    fm_bias = scal_ref[0, 0]
    wo0 = scal_ref[0, 1]
    bo0 = scal_ref[0, 2]
    y1 = jnp.sum(w1g * v, axis=1, keepdims=True)
    vexp = jnp.dot(v, r_ref[...], preferred_element_type=jnp.float32)
    ev = x * vexp
    s = jnp.dot(ev, g_ref[...], preferred_element_type=jnp.float32)
    sq = jnp.dot(ev * ev, g_ref[...], preferred_element_type=jnp.float32)
    y2 = 0.5 * (jnp.sum(s * s, axis=1, keepdims=True)
                - jnp.sum(sq, axis=1, keepdims=True))
    yfm = y1 + y2 + fm_bias
    h = jnp.maximum(jnp.dot(x, w0_ref[...], preferred_element_type=jnp.float32)
                    + b0_ref[...], 0.0)
    h = jnp.maximum(jnp.dot(h, w1_ref[...], preferred_element_type=jnp.float32)
                    + b1_ref[...], 0.0)
    h = jnp.maximum(jnp.dot(h, w2_ref[...], preferred_element_type=jnp.float32)
                    + b2_ref[...], 0.0)
    z = yfm * wo0 + jnp.dot(h, wo_ref[...], preferred_element_type=jnp.float32) + bo0
    o_ref[...] = jax.nn.sigmoid(z)


def kernel(feat_index, feat_value, emb_W, w1, fm_bias, W0, b0, W1, b1, W2, b2, Wo, bo):
    idx_flat = feat_index.reshape(-1).astype(jnp.int32)
    planes = pl.pallas_call(
        _detile_body,
        grid=(2, NBD),
        in_specs=[pl.BlockSpec((8, CPD), lambda g, i: (g, i))],
        out_specs=pl.BlockSpec((8, CPD // 128, 128), lambda g, i: (g, i, 0)),
        out_shape=jax.ShapeDtypeStruct((16, PS, 128), jnp.float32),
    )(emb_W.T)
    w1_128 = pl.pallas_call(
        _w1_body,
        grid=(2,),
        in_specs=[pl.BlockSpec((1, W1C), lambda i: (0, i))],
        out_specs=pl.BlockSpec((W1C // 128, 128), lambda i: (i, 0)),
        out_shape=jax.ShapeDtypeStruct((W1_ROWS, 128), jnp.float32),
    )(w1.T)
    table1d = _sc_transpose_fn()(planes.reshape(16 * PS * 128))
    emb_lin = table1d.reshape(VT, E)
    emb_rows, w1g = _sc_gather_fn()(idx_flat, emb_lin, w1_128.reshape(W1_LIN))
    x = emb_rows.reshape(B, D_IN)
    w1g2 = w1g.reshape(B, F)
    scal = jnp.stack([fm_bias.astype(jnp.float32), Wo[0, 0], bo[0]]).reshape(1, 3)
    out = pl.pallas_call(
        _tc_body,
        grid=(GRID,),
        in_specs=[
            pl.BlockSpec((BM, D_IN), lambda i: (i, 0)),
            pl.BlockSpec((BM, F), lambda i: (i, 0)),
            pl.BlockSpec((BM, F), lambda i: (i, 0)),
            pl.BlockSpec((F, D_IN), lambda i: (0, 0)),
            pl.BlockSpec((D_IN, E), lambda i: (0, 0)),
            pl.BlockSpec((D_IN, 32), lambda i: (0, 0)),
            pl.BlockSpec((1, 32), lambda i: (0, 0)),
            pl.BlockSpec((32, 32), lambda i: (0, 0)),
            pl.BlockSpec((1, 32), lambda i: (0, 0)),
            pl.BlockSpec((32, 32), lambda i: (0, 0)),
            pl.BlockSpec((1, 32), lambda i: (0, 0)),
            pl.BlockSpec((32, 1), lambda i: (0, 0)),
            pl.BlockSpec((1, 3), lambda i: (0, 0)),
        ],
        out_specs=pl.BlockSpec((BM, 1), lambda i: (i, 0)),
        out_shape=jax.ShapeDtypeStruct((B, 1), jnp.float32),
    )(x, feat_value, w1g2, jnp.asarray(_R_np), jnp.asarray(_G_np),
      W0, b0.reshape(1, 32), W1, b1.reshape(1, 32), W2, b2.reshape(1, 32),
      Wo[1:, :], scal)
    return out


# speedup vs baseline: 1.4769x; 1.0715x over previous
import functools

import numpy as np
import jax
import jax.numpy as jnp
from jax import lax
from jax.experimental import pallas as pl
from jax.experimental.pallas import tpu as pltpu
from jax.experimental.pallas import tpu_sc as plsc

B = 16384
F = 26
E = 16
FEAT_DIM = 1000000
D_IN = F * E
BF = B * F
NW = 32

CPD = 32768
NBD = -(-FEAT_DIM // CPD)
PS = NBD * CPD // 128
PSW = PS * 128


def _detile_body(xt_ref, o_ref):
    o_ref[...] = xt_ref[...].reshape(8, CPD // 128, 128)


W1C = 524288
W1_ROWS = 2 * W1C // 128
W1_LIN = W1_ROWS * 128


def _w1_body(w_ref, o_ref):
    o_ref[...] = w_ref[...].reshape(W1C // 128, 128)


CV = 1664
CH_PER_W = 19
NCH = NW * CH_PER_W
VT = NCH * CV
assert VT <= PSW


@functools.lru_cache(maxsize=None)
def _sc_transpose_fn():
    mesh = plsc.VectorSubcoreMesh(core_axis_name="c", subcore_axis_name="s")

    @functools.partial(
        pl.kernel,
        mesh=mesh,
        compiler_params=pltpu.CompilerParams(use_tc_tiling_on_sc=False,
                                             needs_layout_passes=False),
        out_type=jax.ShapeDtypeStruct((VT * E,), jnp.float32),
        scratch_types=[
            pltpu.VMEM((E * CV,), jnp.float32),
            pltpu.VMEM((E * CV,), jnp.float32),
            pltpu.VMEM((CV * E,), jnp.float32),
            pltpu.VMEM((CV * E,), jnp.float32),
            pltpu.SemaphoreType.DMA,
            pltpu.SemaphoreType.DMA,
        ],
    )
    def _sc_transpose(planes_hbm, out_hbm, strips_a, strips_b, o16_a, o16_b,
                      sem_a, sem_b):
        wid = lax.axis_index("s") * 2 + lax.axis_index("c")
        base16 = lax.iota(jnp.int32, 16) * CV
        strips = (strips_a, strips_b)
        o16s = (o16_a, o16_b)
        sems = (sem_a, sem_b)

        def start_loads(t):
            v0 = (wid + NW * t) * CV
            buf = strips[t % 2]
            return [
                pltpu.async_copy(planes_hbm.at[pl.ds(e * PSW + v0, CV)],
                                 buf.at[pl.ds(e * CV, CV)], sems[t % 2])
                for e in range(E)
            ]

        pend = start_loads(0)
        for t in range(CH_PER_W):
            for h in pend:
                h.wait()
            if t + 1 < CH_PER_W:
                pend = start_loads(t + 1)
            sbuf = strips[t % 2]
            obuf = o16s[t % 2]

            @plsc.parallel_loop(0, CV, 1, unroll=16)
            def _row(i):
                col = plsc.load_gather(sbuf, [base16 + i])
                obuf[pl.ds(i * 16, 16)] = col

            v0 = (wid + NW * t) * CV
            pltpu.sync_copy(obuf, out_hbm.at[pl.ds(v0 * E, CV * E)])

    return _sc_transpose


PER_W = BF // NW
K = 1664
NCHUNK = PER_W // K


@functools.lru_cache(maxsize=None)
def _sc_gather_fn():
    mesh = plsc.VectorSubcoreMesh(core_axis_name="c", subcore_axis_name="s")

    @functools.partial(
        pl.kernel,
        mesh=mesh,
        compiler_params=pltpu.CompilerParams(use_tc_tiling_on_sc=False),
        out_type=(
            jax.ShapeDtypeStruct((BF, E), jnp.float32),
            jax.ShapeDtypeStruct((BF,), jnp.float32),
        ),
        scratch_types=[
            pltpu.VMEM((K,), jnp.int32),
            pltpu.VMEM((K,), jnp.int32),
            pltpu.VMEM((K, E), jnp.float32),
            pltpu.VMEM((K, E), jnp.float32),
            pltpu.VMEM((K,), jnp.float32),
            pltpu.VMEM((K,), jnp.float32),
            pltpu.SemaphoreType.DMA,
            pltpu.SemaphoreType.DMA,
        ],
    )
    def _sc_gather(idx_hbm, emb_hbm, w1_hbm, emb_out, w1_out,
                   idx_a, idx_b, rows_a, rows_b, w1a, w1b, sem_r, sem_w):
        wid = lax.axis_index("s") * 2 + lax.axis_index("c")
        base = wid * PER_W
        idx_bufs = (idx_a, idx_b)
        row_bufs = (rows_a, rows_b)
        w1_bufs = (w1a, w1b)

        def load_idx(c):
            pltpu.sync_copy(idx_hbm.at[pl.ds(base + c * K, K)], idx_bufs[c % 2])

        def start_gather(c):
            h1 = pltpu.async_copy(emb_hbm.at[idx_bufs[c % 2]], row_bufs[c % 2], sem_r)
            h2 = pltpu.async_copy(w1_hbm.at[idx_bufs[c % 2]], w1_bufs[c % 2], sem_w)
            return (h1, h2)

        load_idx(0)
        pend = start_gather(0)
        for c in range(NCHUNK):
            if c + 1 < NCHUNK:
                load_idx(c + 1)
            for h in pend:
                h.wait()
            if c + 1 < NCHUNK:
                pend = start_gather(c + 1)
            pltpu.sync_copy(row_bufs[c % 2], emb_out.at[pl.ds(base + c * K, K)])
            pltpu.sync_copy(w1_bufs[c % 2], w1_out.at[pl.ds(base + c * K, K)])

    return _sc_gather


_R_np = np.kron(np.eye(F, dtype=np.float32), np.ones((1, E), dtype=np.float32))
_G_np = np.tile(np.eye(E, dtype=np.float32), (F, 1))

BM = 1024
GRID = B // BM


def _tc_body(x_ref, v_ref, w1g_ref, r_ref, g_ref, w0_ref, b0_ref, w1_ref, b1_ref,
             w2_ref, b2_ref, wo_ref, scal_ref, o_ref):
    x = x_ref[...]
    v = v_ref[...]
    w1g = w1g_ref[...]
    fm_bias = scal_ref[0, 0]
    wo0 = scal_ref[0, 1]
    bo0 = scal_ref[0, 2]
    y1 = jnp.sum(w1g * v, axis=1, keepdims=True)
    vexp = jnp.dot(v, r_ref[...], preferred_element_type=jnp.float32)
    ev = x * vexp
    s = jnp.dot(ev, g_ref[...], preferred_element_type=jnp.float32)
    sq = jnp.dot(ev * ev, g_ref[...], preferred_element_type=jnp.float32)
    y2 = 0.5 * (jnp.sum(s * s, axis=1, keepdims=True)
                - jnp.sum(sq, axis=1, keepdims=True))
    yfm = y1 + y2 + fm_bias
    h = jnp.maximum(jnp.dot(x, w0_ref[...], preferred_element_type=jnp.float32)
                    + b0_ref[...], 0.0)
    h = jnp.maximum(jnp.dot(h, w1_ref[...], preferred_element_type=jnp.float32)
                    + b1_ref[...], 0.0)
    h = jnp.maximum(jnp.dot(h, w2_ref[...], preferred_element_type=jnp.float32)
                    + b2_ref[...], 0.0)
    z = yfm * wo0 + jnp.dot(h, wo_ref[...], preferred_element_type=jnp.float32) + bo0
    o_ref[...] = jax.nn.sigmoid(z)


def kernel(feat_index, feat_value, emb_W, w1, fm_bias, W0, b0, W1, b1, W2, b2, Wo, bo):
    idx_flat = feat_index.reshape(-1).astype(jnp.int32)
    planes = pl.pallas_call(
        _detile_body,
        grid=(2, NBD),
        in_specs=[pl.BlockSpec((8, CPD), lambda g, i: (g, i))],
        out_specs=pl.BlockSpec((8, CPD // 128, 128), lambda g, i: (g, i, 0)),
        out_shape=jax.ShapeDtypeStruct((16, PS, 128), jnp.float32),
    )(emb_W.T)
    w1_128 = pl.pallas_call(
        _w1_body,
        grid=(2,),
        in_specs=[pl.BlockSpec((1, W1C), lambda i: (0, i))],
        out_specs=pl.BlockSpec((W1C // 128, 128), lambda i: (i, 0)),
        out_shape=jax.ShapeDtypeStruct((W1_ROWS, 128), jnp.float32),
    )(w1.T)
    table1d = _sc_transpose_fn()(planes.reshape(16 * PS * 128))
    emb_lin = table1d.reshape(VT, E)
    emb_rows, w1g = _sc_gather_fn()(idx_flat, emb_lin, w1_128.reshape(W1_LIN))
    x = emb_rows.reshape(B, D_IN)
    w1g2 = w1g.reshape(B, F)
    scal = jnp.stack([fm_bias.astype(jnp.float32), Wo[0, 0], bo[0]]).reshape(1, 3)
    out = pl.pallas_call(
        _tc_body,
        grid=(GRID,),
        in_specs=[
            pl.BlockSpec((BM, D_IN), lambda i: (i, 0)),
            pl.BlockSpec((BM, F), lambda i: (i, 0)),
            pl.BlockSpec((BM, F), lambda i: (i, 0)),
            pl.BlockSpec((F, D_IN), lambda i: (0, 0)),
            pl.BlockSpec((D_IN, E), lambda i: (0, 0)),
            pl.BlockSpec((D_IN, 32), lambda i: (0, 0)),
            pl.BlockSpec((1, 32), lambda i: (0, 0)),
            pl.BlockSpec((32, 32), lambda i: (0, 0)),
            pl.BlockSpec((1, 32), lambda i: (0, 0)),
            pl.BlockSpec((32, 32), lambda i: (0, 0)),
            pl.BlockSpec((1, 32), lambda i: (0, 0)),
            pl.BlockSpec((32, 1), lambda i: (0, 0)),
            pl.BlockSpec((1, 3), lambda i: (0, 0)),
        ],
        out_specs=pl.BlockSpec((BM, 1), lambda i: (i, 0)),
        out_shape=jax.ShapeDtypeStruct((B, 1), jnp.float32),
    )(x, feat_value, w1g2, jnp.asarray(_R_np), jnp.asarray(_G_np),
      W0, b0.reshape(1, 32), W1, b1.reshape(1, 32), W2, b2.reshape(1, 32),
      Wo[1:, :], scal)
    return out


# speedup vs baseline: 1.5371x; 1.0407x over previous
import functools

import numpy as np
import jax
import jax.numpy as jnp
from jax import lax
from jax.experimental import pallas as pl
from jax.experimental.pallas import tpu as pltpu
from jax.experimental.pallas import tpu_sc as plsc

B = 16384
F = 26
E = 16
FEAT_DIM = 1000000
D_IN = F * E
BF = B * F
NW = 32

CPD = 32768
NBD = -(-FEAT_DIM // CPD)
PS = NBD * CPD // 128
PSW = PS * 128


def _detile_body(xt_ref, o_ref):
    o_ref[...] = xt_ref[...].reshape(8, CPD // 128, 128)


W1C = 524288
W1_ROWS = 2 * W1C // 128
W1_LIN = W1_ROWS * 128


def _w1_body(w_ref, o_ref):
    o_ref[...] = w_ref[...].reshape(W1C // 128, 128)


CV = 1664
CH_PER_W = 19
NCH = NW * CH_PER_W
VT = NCH * CV
assert VT <= PSW


@functools.lru_cache(maxsize=None)
def _sc_transpose_fn():
    mesh = plsc.VectorSubcoreMesh(core_axis_name="c", subcore_axis_name="s")

    @functools.partial(
        pl.kernel,
        mesh=mesh,
        compiler_params=pltpu.CompilerParams(use_tc_tiling_on_sc=False,
                                             needs_layout_passes=False),
        out_type=jax.ShapeDtypeStruct((VT * E,), jnp.float32),
        scratch_types=[
            pltpu.VMEM((E * CV,), jnp.float32),
            pltpu.VMEM((E * CV,), jnp.float32),
            pltpu.VMEM((CV * E,), jnp.float32),
            pltpu.VMEM((CV * E,), jnp.float32),
            pltpu.SemaphoreType.DMA,
            pltpu.SemaphoreType.DMA,
            pltpu.SemaphoreType.DMA,
            pltpu.SemaphoreType.DMA,
        ],
    )
    def _sc_transpose(planes_hbm, out_hbm, strips_a, strips_b, o16_a, o16_b,
                      sem_a, sem_b, wsem_a, wsem_b):
        wid = lax.axis_index("s") * 2 + lax.axis_index("c")
        base16 = lax.iota(jnp.int32, 16) * CV
        strips = (strips_a, strips_b)
        o16s = (o16_a, o16_b)
        sems = (sem_a, sem_b)

        def start_loads(t):
            v0 = (wid + NW * t) * CV
            buf = strips[t % 2]
            return [
                pltpu.async_copy(planes_hbm.at[pl.ds(e * PSW + v0, CV)],
                                 buf.at[pl.ds(e * CV, CV)], sems[t % 2])
                for e in range(E)
            ]

        wsems = (wsem_a, wsem_b)
        pend = start_loads(0)
        wr = [None, None]
        for t in range(CH_PER_W):
            for h in pend:
                h.wait()
            if t + 1 < CH_PER_W:
                pend = start_loads(t + 1)
            sbuf = strips[t % 2]
            obuf = o16s[t % 2]
            if wr[t % 2] is not None:
                wr[t % 2].wait()

            @plsc.parallel_loop(0, CV, 1, unroll=16)
            def _row(i):
                col = plsc.load_gather(sbuf, [base16 + i])
                obuf[pl.ds(i * 16, 16)] = col

            v0 = (wid + NW * t) * CV
            wr[t % 2] = pltpu.async_copy(
                obuf, out_hbm.at[pl.ds(v0 * E, CV * E)], wsems[t % 2])
        for h in wr:
            if h is not None:
                h.wait()

    return _sc_transpose


PER_W = BF // NW
K = 1664
NCHUNK = PER_W // K


@functools.lru_cache(maxsize=None)
def _sc_gather_fn():
    mesh = plsc.VectorSubcoreMesh(core_axis_name="c", subcore_axis_name="s")

    @functools.partial(
        pl.kernel,
        mesh=mesh,
        compiler_params=pltpu.CompilerParams(use_tc_tiling_on_sc=False),
        out_type=(
            jax.ShapeDtypeStruct((BF, E), jnp.float32),
            jax.ShapeDtypeStruct((BF,), jnp.float32),
        ),
        scratch_types=[
            pltpu.VMEM((K,), jnp.int32),
            pltpu.VMEM((K,), jnp.int32),
            pltpu.VMEM((K, E), jnp.float32),
            pltpu.VMEM((K, E), jnp.float32),
            pltpu.VMEM((K,), jnp.float32),
            pltpu.VMEM((K,), jnp.float32),
            pltpu.SemaphoreType.DMA,
            pltpu.SemaphoreType.DMA,
        ],
    )
    def _sc_gather(idx_hbm, emb_hbm, w1_hbm, emb_out, w1_out,
                   idx_a, idx_b, rows_a, rows_b, w1a, w1b, sem_r, sem_w):
        wid = lax.axis_index("s") * 2 + lax.axis_index("c")
        base = wid * PER_W
        idx_bufs = (idx_a, idx_b)
        row_bufs = (rows_a, rows_b)
        w1_bufs = (w1a, w1b)

        def load_idx(c):
            pltpu.sync_copy(idx_hbm.at[pl.ds(base + c * K, K)], idx_bufs[c % 2])

        def start_gather(c):
            h1 = pltpu.async_copy(emb_hbm.at[idx_bufs[c % 2]], row_bufs[c % 2], sem_r)
            h2 = pltpu.async_copy(w1_hbm.at[idx_bufs[c % 2]], w1_bufs[c % 2], sem_w)
            return (h1, h2)

        load_idx(0)
        pend = start_gather(0)
        for c in range(NCHUNK):
            if c + 1 < NCHUNK:
                load_idx(c + 1)
            for h in pend:
                h.wait()
            if c + 1 < NCHUNK:
                pend = start_gather(c + 1)
            pltpu.sync_copy(row_bufs[c % 2], emb_out.at[pl.ds(base + c * K, K)])
            pltpu.sync_copy(w1_bufs[c % 2], w1_out.at[pl.ds(base + c * K, K)])

    return _sc_gather


_R_np = np.kron(np.eye(F, dtype=np.float32), np.ones((1, E), dtype=np.float32))
_G_np = np.tile(np.eye(E, dtype=np.float32), (F, 1))

BM = 1024
GRID = B // BM


def _tc_body(x_ref, v_ref, w1g_ref, r_ref, g_ref, w0_ref, b0_ref, w1_ref, b1_ref,
             w2_ref, b2_ref, wo_ref, scal_ref, o_ref):
    x = x_ref[...]
    v = v_ref[...]
    w1g = w1g_ref[...]
    fm_bias = scal_ref[0, 0]
    wo0 = scal_ref[0, 1]
    bo0 = scal_ref[0, 2]
    y1 = jnp.sum(w1g * v, axis=1, keepdims=True)
    vexp = jnp.dot(v, r_ref[...], preferred_element_type=jnp.float32)
    ev = x * vexp
    s = jnp.dot(ev, g_ref[...], preferred_element_type=jnp.float32)
    sq = jnp.dot(ev * ev, g_ref[...], preferred_element_type=jnp.float32)
    y2 = 0.5 * (jnp.sum(s * s, axis=1, keepdims=True)
                - jnp.sum(sq, axis=1, keepdims=True))
    yfm = y1 + y2 + fm_bias
    h = jnp.maximum(jnp.dot(x, w0_ref[...], preferred_element_type=jnp.float32)
                    + b0_ref[...], 0.0)
    h = jnp.maximum(jnp.dot(h, w1_ref[...], preferred_element_type=jnp.float32)
                    + b1_ref[...], 0.0)
    h = jnp.maximum(jnp.dot(h, w2_ref[...], preferred_element_type=jnp.float32)
                    + b2_ref[...], 0.0)
    z = yfm * wo0 + jnp.dot(h, wo_ref[...], preferred_element_type=jnp.float32) + bo0
    o_ref[...] = jax.nn.sigmoid(z)


def kernel(feat_index, feat_value, emb_W, w1, fm_bias, W0, b0, W1, b1, W2, b2, Wo, bo):
    idx_flat = feat_index.reshape(-1).astype(jnp.int32)
    planes = pl.pallas_call(
        _detile_body,
        grid=(2, NBD),
        in_specs=[pl.BlockSpec((8, CPD), lambda g, i: (g, i))],
        out_specs=pl.BlockSpec((8, CPD // 128, 128), lambda g, i: (g, i, 0)),
        out_shape=jax.ShapeDtypeStruct((16, PS, 128), jnp.float32),
    )(emb_W.T)
    w1_128 = pl.pallas_call(
        _w1_body,
        grid=(2,),
        in_specs=[pl.BlockSpec((1, W1C), lambda i: (0, i))],
        out_specs=pl.BlockSpec((W1C // 128, 128), lambda i: (i, 0)),
        out_shape=jax.ShapeDtypeStruct((W1_ROWS, 128), jnp.float32),
    )(w1.T)
    table1d = _sc_transpose_fn()(planes.reshape(16 * PS * 128))
    emb_lin = table1d.reshape(VT, E)
    emb_rows, w1g = _sc_gather_fn()(idx_flat, emb_lin, w1_128.reshape(W1_LIN))
    x = emb_rows.reshape(B, D_IN)
    w1g2 = w1g.reshape(B, F)
    scal = jnp.stack([fm_bias.astype(jnp.float32), Wo[0, 0], bo[0]]).reshape(1, 3)
    out = pl.pallas_call(
        _tc_body,
        grid=(GRID,),
        in_specs=[
            pl.BlockSpec((BM, D_IN), lambda i: (i, 0)),
            pl.BlockSpec((BM, F), lambda i: (i, 0)),
            pl.BlockSpec((BM, F), lambda i: (i, 0)),
            pl.BlockSpec((F, D_IN), lambda i: (0, 0)),
            pl.BlockSpec((D_IN, E), lambda i: (0, 0)),
            pl.BlockSpec((D_IN, 32), lambda i: (0, 0)),
            pl.BlockSpec((1, 32), lambda i: (0, 0)),
            pl.BlockSpec((32, 32), lambda i: (0, 0)),
            pl.BlockSpec((1, 32), lambda i: (0, 0)),
            pl.BlockSpec((32, 32), lambda i: (0, 0)),
            pl.BlockSpec((1, 32), lambda i: (0, 0)),
            pl.BlockSpec((32, 1), lambda i: (0, 0)),
            pl.BlockSpec((1, 3), lambda i: (0, 0)),
        ],
        out_specs=pl.BlockSpec((BM, 1), lambda i: (i, 0)),
        out_shape=jax.ShapeDtypeStruct((B, 1), jnp.float32),
    )(x, feat_value, w1g2, jnp.asarray(_R_np), jnp.asarray(_G_np),
      W0, b0.reshape(1, 32), W1, b1.reshape(1, 32), W2, b2.reshape(1, 32),
      Wo[1:, :], scal)
    return out


# speedup vs baseline: 1.5449x; 1.0051x over previous
import functools

import numpy as np
import jax
import jax.numpy as jnp
from jax import lax
from jax.experimental import pallas as pl
from jax.experimental.pallas import tpu as pltpu
from jax.experimental.pallas import tpu_sc as plsc

B = 16384
F = 26
E = 16
FEAT_DIM = 1000000
D_IN = F * E
BF = B * F
NW = 32

CPD = 32768
NBD = -(-FEAT_DIM // CPD)
PS = NBD * CPD // 128
PSW = PS * 128


def _detile_body(xt_ref, o_ref):
    o_ref[...] = xt_ref[...].reshape(8, CPD // 128, 128)


W1C = 524288
W1_ROWS = 2 * W1C // 128
W1_LIN = W1_ROWS * 128


def _w1_body(w_ref, o_ref):
    o_ref[...] = w_ref[...].reshape(W1C // 128, 128)


CV = 1664
CH_PER_W = 19
NCH = NW * CH_PER_W
VT = NCH * CV
assert VT <= PSW


@functools.lru_cache(maxsize=None)
def _sc_transpose_fn():
    mesh = plsc.VectorSubcoreMesh(core_axis_name="c", subcore_axis_name="s")

    @functools.partial(
        pl.kernel,
        mesh=mesh,
        compiler_params=pltpu.CompilerParams(use_tc_tiling_on_sc=False,
                                             needs_layout_passes=False),
        out_type=jax.ShapeDtypeStruct((VT * E,), jnp.float32),
        scratch_types=[
            pltpu.VMEM((E * CV,), jnp.float32),
            pltpu.VMEM((E * CV,), jnp.float32),
            pltpu.VMEM((CV * E,), jnp.float32),
            pltpu.VMEM((CV * E,), jnp.float32),
            pltpu.SemaphoreType.DMA,
            pltpu.SemaphoreType.DMA,
            pltpu.SemaphoreType.DMA,
            pltpu.SemaphoreType.DMA,
        ],
    )
    def _sc_transpose(planes_hbm, out_hbm, strips_a, strips_b, o16_a, o16_b,
                      sem_a, sem_b, wsem_a, wsem_b):
        wid = lax.axis_index("s") * 2 + lax.axis_index("c")
        base16 = lax.iota(jnp.int32, 16) * CV
        strips = (strips_a, strips_b)
        o16s = (o16_a, o16_b)
        sems = (sem_a, sem_b)

        def start_loads(t):
            v0 = (wid + NW * t) * CV
            buf = strips[t % 2]
            return [
                pltpu.async_copy(planes_hbm.at[pl.ds(e * PSW + v0, CV)],
                                 buf.at[pl.ds(e * CV, CV)], sems[t % 2])
                for e in range(E)
            ]

        wsems = (wsem_a, wsem_b)
        pend = start_loads(0)
        wr = [None, None]
        for t in range(CH_PER_W):
            for h in pend:
                h.wait()
            if t + 1 < CH_PER_W:
                pend = start_loads(t + 1)
            sbuf = strips[t % 2]
            obuf = o16s[t % 2]
            if wr[t % 2] is not None:
                wr[t % 2].wait()

            @plsc.parallel_loop(0, CV, 1, unroll=32)
            def _row(i):
                col = plsc.load_gather(sbuf, [base16 + i])
                obuf[pl.ds(i * 16, 16)] = col

            v0 = (wid + NW * t) * CV
            wr[t % 2] = pltpu.async_copy(
                obuf, out_hbm.at[pl.ds(v0 * E, CV * E)], wsems[t % 2])
        for h in wr:
            if h is not None:
                h.wait()

    return _sc_transpose


PER_W = BF // NW
K = 1664
NCHUNK = PER_W // K


@functools.lru_cache(maxsize=None)
def _sc_gather_fn():
    mesh = plsc.VectorSubcoreMesh(core_axis_name="c", subcore_axis_name="s")

    @functools.partial(
        pl.kernel,
        mesh=mesh,
        compiler_params=pltpu.CompilerParams(use_tc_tiling_on_sc=False),
        out_type=(
            jax.ShapeDtypeStruct((BF, E), jnp.float32),
            jax.ShapeDtypeStruct((BF,), jnp.float32),
        ),
        scratch_types=[
            pltpu.VMEM((K,), jnp.int32),
            pltpu.VMEM((K,), jnp.int32),
            pltpu.VMEM((K, E), jnp.float32),
            pltpu.VMEM((K, E), jnp.float32),
            pltpu.VMEM((K,), jnp.float32),
            pltpu.VMEM((K,), jnp.float32),
            pltpu.SemaphoreType.DMA,
            pltpu.SemaphoreType.DMA,
        ],
    )
    def _sc_gather(idx_hbm, emb_hbm, w1_hbm, emb_out, w1_out,
                   idx_a, idx_b, rows_a, rows_b, w1a, w1b, sem_r, sem_w):
        wid = lax.axis_index("s") * 2 + lax.axis_index("c")
        base = wid * PER_W
        idx_bufs = (idx_a, idx_b)
        row_bufs = (rows_a, rows_b)
        w1_bufs = (w1a, w1b)

        def load_idx(c):
            pltpu.sync_copy(idx_hbm.at[pl.ds(base + c * K, K)], idx_bufs[c % 2])

        def start_gather(c):
            h1 = pltpu.async_copy(emb_hbm.at[idx_bufs[c % 2]], row_bufs[c % 2], sem_r)
            h2 = pltpu.async_copy(w1_hbm.at[idx_bufs[c % 2]], w1_bufs[c % 2], sem_w)
            return (h1, h2)

        load_idx(0)
        pend = start_gather(0)
        for c in range(NCHUNK):
            if c + 1 < NCHUNK:
                load_idx(c + 1)
            for h in pend:
                h.wait()
            if c + 1 < NCHUNK:
                pend = start_gather(c + 1)
            pltpu.sync_copy(row_bufs[c % 2], emb_out.at[pl.ds(base + c * K, K)])
            pltpu.sync_copy(w1_bufs[c % 2], w1_out.at[pl.ds(base + c * K, K)])

    return _sc_gather


_R_np = np.kron(np.eye(F, dtype=np.float32), np.ones((1, E), dtype=np.float32))
_G_np = np.tile(np.eye(E, dtype=np.float32), (F, 1))

BM = 2048
GRID = B // BM


def _tc_body(x_ref, v_ref, w1g_ref, r_ref, g_ref, w0_ref, b0_ref, w1_ref, b1_ref,
             w2_ref, b2_ref, wo_ref, scal_ref, o_ref):
    x = x_ref[...]
    v = v_ref[...]
    w1g = w1g_ref[...]
    fm_bias = scal_ref[0, 0]
    wo0 = scal_ref[0, 1]
    bo0 = scal_ref[0, 2]
    y1 = jnp.sum(w1g * v, axis=1, keepdims=True)
    vexp = jnp.dot(v, r_ref[...], preferred_element_type=jnp.float32)
    ev = x * vexp
    s = jnp.dot(ev, g_ref[...], preferred_element_type=jnp.float32)
    sq = jnp.dot(ev * ev, g_ref[...], preferred_element_type=jnp.float32)
    y2 = 0.5 * (jnp.sum(s * s, axis=1, keepdims=True)
                - jnp.sum(sq, axis=1, keepdims=True))
    yfm = y1 + y2 + fm_bias
    h = jnp.maximum(jnp.dot(x, w0_ref[...], preferred_element_type=jnp.float32)
                    + b0_ref[...], 0.0)
    h = jnp.maximum(jnp.dot(h, w1_ref[...], preferred_element_type=jnp.float32)
                    + b1_ref[...], 0.0)
    h = jnp.maximum(jnp.dot(h, w2_ref[...], preferred_element_type=jnp.float32)
                    + b2_ref[...], 0.0)
    z = yfm * wo0 + jnp.dot(h, wo_ref[...], preferred_element_type=jnp.float32) + bo0
    o_ref[...] = jax.nn.sigmoid(z)


def kernel(feat_index, feat_value, emb_W, w1, fm_bias, W0, b0, W1, b1, W2, b2, Wo, bo):
    idx_flat = feat_index.reshape(-1).astype(jnp.int32)
    planes = pl.pallas_call(
        _detile_body,
        grid=(2, NBD),
        in_specs=[pl.BlockSpec((8, CPD), lambda g, i: (g, i))],
        out_specs=pl.BlockSpec((8, CPD // 128, 128), lambda g, i: (g, i, 0)),
        out_shape=jax.ShapeDtypeStruct((16, PS, 128), jnp.float32),
    )(emb_W.T)
    w1_128 = pl.pallas_call(
        _w1_body,
        grid=(2,),
        in_specs=[pl.BlockSpec((1, W1C), lambda i: (0, i))],
        out_specs=pl.BlockSpec((W1C // 128, 128), lambda i: (i, 0)),
        out_shape=jax.ShapeDtypeStruct((W1_ROWS, 128), jnp.float32),
    )(w1.T)
    table1d = _sc_transpose_fn()(planes.reshape(16 * PS * 128))
    emb_lin = table1d.reshape(VT, E)
    emb_rows, w1g = _sc_gather_fn()(idx_flat, emb_lin, w1_128.reshape(W1_LIN))
    x = emb_rows.reshape(B, D_IN)
    w1g2 = w1g.reshape(B, F)
    scal = jnp.stack([fm_bias.astype(jnp.float32), Wo[0, 0], bo[0]]).reshape(1, 3)
    out = pl.pallas_call(
        _tc_body,
        grid=(GRID,),
        in_specs=[
            pl.BlockSpec((BM, D_IN), lambda i: (i, 0)),
            pl.BlockSpec((BM, F), lambda i: (i, 0)),
            pl.BlockSpec((BM, F), lambda i: (i, 0)),
            pl.BlockSpec((F, D_IN), lambda i: (0, 0)),
            pl.BlockSpec((D_IN, E), lambda i: (0, 0)),
            pl.BlockSpec((D_IN, 32), lambda i: (0, 0)),
            pl.BlockSpec((1, 32), lambda i: (0, 0)),
            pl.BlockSpec((32, 32), lambda i: (0, 0)),
            pl.BlockSpec((1, 32), lambda i: (0, 0)),
            pl.BlockSpec((32, 32), lambda i: (0, 0)),
            pl.BlockSpec((1, 32), lambda i: (0, 0)),
            pl.BlockSpec((32, 1), lambda i: (0, 0)),
            pl.BlockSpec((1, 3), lambda i: (0, 0)),
        ],
        out_specs=pl.BlockSpec((BM, 1), lambda i: (i, 0)),
        out_shape=jax.ShapeDtypeStruct((B, 1), jnp.float32),
    )(x, feat_value, w1g2, jnp.asarray(_R_np), jnp.asarray(_G_np),
      W0, b0.reshape(1, 32), W1, b1.reshape(1, 32), W2, b2.reshape(1, 32),
      Wo[1:, :], scal)
    return out
